# baseline (speedup 1.0000x reference)
import functools

import jax
import jax.numpy as jnp
from jax import lax
from jax.experimental import pallas as pl
from jax.experimental.pallas import tpu as pltpu
from jax.experimental.pallas import tpu_sc as plsc

EMBEDDING_DIM = 128
MAX_LEAD_TIME = 72
BATCH = 16384
LANES = 16
OUT_CHUNK = 256
UNROLL = 4


def kernel(lead_times, pe):
    info = plsc.get_sparse_core_info()
    num_cores, num_subcores = info.num_cores, info.num_subcores
    num_workers = num_cores * num_subcores
    b_per_w = BATCH // num_workers
    n_chunks = b_per_w // OUT_CHUNK
    vregs_per_row = EMBEDDING_DIM // LANES

    mesh = plsc.VectorSubcoreMesh(core_axis_name="c", subcore_axis_name="s")

    @functools.partial(
        pl.kernel,
        mesh=mesh,
        out_type=jax.ShapeDtypeStruct((BATCH, EMBEDDING_DIM), jnp.float32),
        scratch_types=[
            pltpu.VMEM((MAX_LEAD_TIME + 1, EMBEDDING_DIM), jnp.float32),
            pltpu.VMEM((b_per_w, EMBEDDING_DIM), jnp.float32),
            pltpu.VMEM((b_per_w + LANES,), jnp.int32),
            pltpu.SemaphoreType.DMA,
            pltpu.SemaphoreType.DMA,
        ],
    )
    def emb_kernel(
        lt_hbm, pe_hbm, out_hbm, pe_v, rows_v, idx_v, sem_in, sem_out
    ):
        wid = lax.axis_index("s") * num_cores + lax.axis_index("c")
        base = wid * b_per_w
        cp_tab = pltpu.async_copy(pe_hbm, pe_v, sem_in)
        pltpu.sync_copy(lt_hbm.at[pl.ds(base, b_per_w)], idx_v.at[pl.ds(0, b_per_w)])
        cp_tab.wait()

        @plsc.parallel_loop(0, b_per_w // LANES)
        def _(g):
            v = idx_v[pl.ds(g * LANES, LANES)]
            idx_v[pl.ds(g * LANES, LANES)] = jnp.minimum(
                jnp.maximum(v, 0), MAX_LEAD_TIME
            )

        def chunk_body(c, carry):
            @plsc.parallel_loop(0, OUT_CHUNK, unroll=8)
            def _(b):
                row = c * OUT_CHUNK + b
                r = idx_v[pl.ds(row, LANES)][0]
                for j in range(vregs_per_row):
                    rows_v[row, pl.ds(j * LANES, LANES)] = pe_v[
                        r, pl.ds(j * LANES, LANES)
                    ]

            pltpu.async_copy(
                rows_v.at[pl.ds(c * OUT_CHUNK, OUT_CHUNK)],
                out_hbm.at[pl.ds(base + c * OUT_CHUNK, OUT_CHUNK)],
                sem_out,
            )
            return carry

        lax.fori_loop(0, n_chunks, chunk_body, 0)
        pltpu.make_async_copy(
            out_hbm.at[pl.ds(base, b_per_w)], rows_v, sem_out
        ).wait()

    if lead_times.dtype != jnp.int32:
        lead_times = lead_times.astype(jnp.int32)
    return emb_kernel(lead_times, pe)


# speedup vs baseline: 1.0313x; 1.0313x over previous
import functools

import jax
import jax.numpy as jnp
from jax import lax
from jax.experimental import pallas as pl
from jax.experimental.pallas import tpu as pltpu
from jax.experimental.pallas import tpu_sc as plsc

EMBEDDING_DIM = 128
MAX_LEAD_TIME = 72
BATCH = 16384
LANES = 16
OUT_CHUNK = 128
UNROLL = 4


def kernel(lead_times, pe):
    info = plsc.get_sparse_core_info()
    num_cores, num_subcores = info.num_cores, info.num_subcores
    num_workers = num_cores * num_subcores
    b_per_w = BATCH // num_workers
    n_chunks = b_per_w // OUT_CHUNK
    vregs_per_row = EMBEDDING_DIM // LANES

    mesh = plsc.VectorSubcoreMesh(core_axis_name="c", subcore_axis_name="s")

    @functools.partial(
        pl.kernel,
        mesh=mesh,
        out_type=jax.ShapeDtypeStruct((BATCH, EMBEDDING_DIM), jnp.float32),
        scratch_types=[
            pltpu.VMEM((MAX_LEAD_TIME + 1, EMBEDDING_DIM), jnp.float32),
            pltpu.VMEM((b_per_w, EMBEDDING_DIM), jnp.float32),
            pltpu.VMEM((b_per_w + LANES,), jnp.int32),
            pltpu.SemaphoreType.DMA,
            pltpu.SemaphoreType.DMA,
        ],
    )
    def emb_kernel(
        lt_hbm, pe_hbm, out_hbm, pe_v, rows_v, idx_v, sem_in, sem_out
    ):
        wid = lax.axis_index("c") * num_subcores + lax.axis_index("s")
        base = wid * b_per_w
        cp_tab = pltpu.async_copy(pe_hbm, pe_v, sem_in)
        pltpu.sync_copy(lt_hbm.at[pl.ds(base, b_per_w)], idx_v.at[pl.ds(0, b_per_w)])
        cp_tab.wait()

        @plsc.parallel_loop(0, b_per_w // LANES)
        def _(g):
            v = idx_v[pl.ds(g * LANES, LANES)]
            idx_v[pl.ds(g * LANES, LANES)] = jnp.minimum(
                jnp.maximum(v, 0), MAX_LEAD_TIME
            )

        def chunk_body(c, carry):
            @plsc.parallel_loop(0, OUT_CHUNK, unroll=8)
            def _(b):
                row = c * OUT_CHUNK + b
                r = idx_v[pl.ds(row, LANES)][0]
                for j in range(vregs_per_row):
                    rows_v[row, pl.ds(j * LANES, LANES)] = pe_v[
                        r, pl.ds(j * LANES, LANES)
                    ]

            pltpu.async_copy(
                rows_v.at[pl.ds(c * OUT_CHUNK, OUT_CHUNK)],
                out_hbm.at[pl.ds(base + c * OUT_CHUNK, OUT_CHUNK)],
                sem_out,
            )
            return carry

        lax.fori_loop(0, n_chunks, chunk_body, 0)
        pltpu.make_async_copy(
            out_hbm.at[pl.ds(base, b_per_w)], rows_v, sem_out
        ).wait()

    if lead_times.dtype != jnp.int32:
        lead_times = lead_times.astype(jnp.int32)
    return emb_kernel(lead_times, pe)
